# baseline (speedup 1.0000x reference)
import jax
import jax.numpy as jnp
from jax import lax
from jax.experimental import pallas as pl
from jax.experimental.pallas import tpu as pltpu, tpu_sc as plsc

NUM_CORES = 2
NUM_SUBCORES = 16
LANES = 16
NW = NUM_CORES * NUM_SUBCORES
BATCH = 16384
EMBED = 16
B_PER_W = BATCH // NW
CHUNK = 128
N_CHUNKS = B_PER_W // CHUNK
BLOCKS = B_PER_W // LANES


def _body(uidx_hbm, midx_hbm, ue_hbm, ub_hbm, me_hbm, mb_hbm, out_hbm,
          uidx_v, midx_v, urows_v, mrows_v, ub_v, mb_v, out_v, sem):
    wid = lax.axis_index("s") * NUM_CORES + lax.axis_index("c")
    cbase = wid * N_CHUNKS

    pltpu.sync_copy(uidx_hbm.at[pl.ds(cbase, N_CHUNKS)], uidx_v)
    pltpu.sync_copy(midx_hbm.at[pl.ds(cbase, N_CHUNKS)], midx_v)

    copies = []
    for c in range(N_CHUNKS):
        r = pl.ds(c * CHUNK, CHUNK)
        copies.append(pltpu.async_copy(ue_hbm.at[uidx_v.at[c]], urows_v.at[r], sem))
        copies.append(pltpu.async_copy(me_hbm.at[midx_v.at[c]], mrows_v.at[r], sem))
        copies.append(pltpu.async_copy(ub_hbm.at[uidx_v.at[c]], ub_v.at[r], sem))
        copies.append(pltpu.async_copy(mb_hbm.at[midx_v.at[c]], mb_v.at[r], sem))
    for cp in copies:
        cp.wait()

    lane = lax.iota(jnp.int32, LANES)

    def blk_body(blk, _):
        rows = blk * LANES + lane
        acc = ub_v[pl.ds(blk * LANES, LANES)] + mb_v[pl.ds(blk * LANES, LANES)]
        for d in range(EMBED):
            dvec = jnp.full((LANES,), d, jnp.int32)
            u = plsc.load_gather(urows_v, [rows, dvec])
            m = plsc.load_gather(mrows_v, [rows, dvec])
            acc = acc + u * m
        out_v[pl.ds(blk * LANES, LANES)] = 1.0 / (1.0 + jnp.exp(-acc))
        return ()

    lax.fori_loop(0, BLOCKS, blk_body, (), unroll=False)

    pltpu.sync_copy(out_v, out_hbm.at[pl.ds(wid * B_PER_W, B_PER_W)])


def kernel(inputs, user_embedding, user_bias, movie_embedding, movie_bias):
    uidx = inputs[:, 0].reshape(BATCH // CHUNK, CHUNK)
    midx = inputs[:, 1].reshape(BATCH // CHUNK, CHUNK)
    mesh = plsc.VectorSubcoreMesh(core_axis_name="c", subcore_axis_name="s",
                                  num_cores=NUM_CORES, num_subcores=NUM_SUBCORES)
    out = pl.kernel(
        _body,
        out_type=jax.ShapeDtypeStruct((BATCH,), jnp.float32),
        mesh=mesh,
        compiler_params=pltpu.CompilerParams(
            use_tc_tiling_on_sc=False, needs_layout_passes=False),
        scratch_types=[
            pltpu.VMEM((N_CHUNKS, CHUNK), jnp.int32),
            pltpu.VMEM((N_CHUNKS, CHUNK), jnp.int32),
            pltpu.VMEM((B_PER_W, EMBED), jnp.float32),
            pltpu.VMEM((B_PER_W, EMBED), jnp.float32),
            pltpu.VMEM((B_PER_W,), jnp.float32),
            pltpu.VMEM((B_PER_W,), jnp.float32),
            pltpu.VMEM((B_PER_W,), jnp.float32),
            pltpu.SemaphoreType.DMA,
        ],
    )(uidx, midx, user_embedding, user_bias.reshape(-1),
      movie_embedding, movie_bias.reshape(-1))
    return out.reshape(BATCH, 1)
